# baseline (speedup 1.0000x reference)
import functools

import jax
import jax.numpy as jnp
from jax import lax
from jax.experimental import pallas as pl
from jax.experimental.pallas import tpu as pltpu
from jax.experimental.pallas import tpu_sc as plsc

_L = 16


def _make_gather_sum(B, C, V, D, NW):
    rows_per_w = B // NW
    NV = C // _L
    NJ = D // _L
    mesh = plsc.VectorSubcoreMesh(core_axis_name="c", subcore_axis_name="s")

    @functools.partial(
        pl.kernel,
        mesh=mesh,
        compiler_params=pltpu.CompilerParams(use_tc_tiling_on_sc=False),
        out_type=jax.ShapeDtypeStruct((B, D), jnp.float32),
        scratch_types=[
            pltpu.VMEM((rows_per_w, C), jnp.int32),
            pltpu.VMEM((2, C, D), jnp.float32),
            pltpu.VMEM((rows_per_w, D), jnp.float32),
            pltpu.SemaphoreType.DMA,
            pltpu.SemaphoreType.DMA,
        ],
    )
    def gather_sum(idx_hbm, table_hbm, feat_hbm, idx_v, bufs, feat_v, s0, s1):
        sems = (s0, s1)
        nc = 2
        wid = lax.axis_index("s") * nc + lax.axis_index("c")
        base = wid * rows_per_w
        pltpu.sync_copy(idx_hbm.at[pl.ds(base, rows_per_w)], idx_v)

        def issue(row, h):
            for j in range(NV):
                ivec = idx_v[row, pl.ds(j * _L, _L)]
                pltpu.async_copy(table_hbm.at[ivec],
                                 bufs.at[h, pl.ds(j * _L, _L)], sems[h])

        issue(0, 0)
        issue(1, 1)

        def step(i, carry):
            for h in range(2):
                row = 2 * i + h
                pltpu.make_async_copy(table_hbm.at[pl.ds(0, C)],
                                      bufs.at[h], sems[h]).wait()
                zero = jnp.zeros((_L,), jnp.float32)

                def acc_body(r, acc, h=h):
                    return tuple(acc[j] + bufs[h, r, pl.ds(j * _L, _L)]
                                 for j in range(NJ))

                acc = plsc.parallel_loop(0, C, 1, unroll=8,
                                         carry=(zero,) * NJ)(acc_body)

                @pl.when(row < rows_per_w - 2)
                def _(row=row, h=h):
                    issue(row + 2, h)

                for j in range(NJ):
                    feat_v[row, pl.ds(j * _L, _L)] = acc[j]
            return carry

        lax.fori_loop(0, rows_per_w // 2, step, 0)
        pltpu.sync_copy(feat_v, feat_hbm.at[pl.ds(base, rows_per_w)])

    return gather_sum


def _linear_body(x_ref, w_ref, b_ref, o_ref):
    o_ref[...] = (
        jnp.dot(x_ref[...], w_ref[...], preferred_element_type=jnp.float32)
        + b_ref[...]
    )


def kernel(text, text_len, table, W, b):
    del text_len
    B, S = text.shape
    V, D = table.shape
    NC = W.shape[0]
    C = S + (-S) % (8 * _L)
    idx = jnp.pad(text, ((0, 0), (0, C - S)))
    info = plsc.get_sparse_core_info()
    NW = info.num_cores * info.num_subcores
    feat = _make_gather_sum(B, C, V, D, NW)(idx, table)
    out = pl.pallas_call(
        _linear_body,
        out_shape=jax.ShapeDtypeStruct((B, NC), jnp.float32),
    )(feat, W.T, b.reshape(1, NC))
    return out


# speedup vs baseline: 1.0273x; 1.0273x over previous
import functools

import jax
import jax.numpy as jnp
from jax import lax
from jax.experimental import pallas as pl
from jax.experimental.pallas import tpu as pltpu
from jax.experimental.pallas import tpu_sc as plsc

_L = 16


def _transpose_body(a_ref, b_ref, out_ref):
    out_ref[...] = jnp.concatenate([a_ref[...].T, b_ref[...].T], axis=1)


def _row_major_table(tableT, V, D):
    CB = 1024
    nblk = -(-(V // 2) // CB)
    H = (nblk - 1) * CB
    NR = nblk * CB
    out = pl.pallas_call(
        _transpose_body,
        grid=(nblk,),
        in_specs=[
            pl.BlockSpec((D, CB), lambda i: (0, i)),
            pl.BlockSpec((D, CB), lambda i, n=nblk: (0, i + n - 1)),
        ],
        out_specs=pl.BlockSpec((CB, 2 * D), lambda i: (i, 0)),
        out_shape=jax.ShapeDtypeStruct((NR, 2 * D), jnp.float32),
    )(tableT, tableT)
    return out, H, NR


def _make_gather_sum(B, C, V, D, NW):
    rows_per_w = B // NW
    pairs_per_w = rows_per_w // 2
    C2 = 2 * C
    NJ = D // _L
    mesh = plsc.VectorSubcoreMesh(core_axis_name="c", subcore_axis_name="s")

    @functools.partial(
        pl.kernel,
        mesh=mesh,
        compiler_params=pltpu.CompilerParams(use_tc_tiling_on_sc=False),
        out_type=jax.ShapeDtypeStruct((B, D), jnp.float32),
        scratch_types=[
            pltpu.VMEM((pairs_per_w, C2), jnp.int32),
            pltpu.VMEM((2, C2, D), jnp.float32),
            pltpu.VMEM((rows_per_w, D), jnp.float32),
            pltpu.SemaphoreType.DMA,
            pltpu.SemaphoreType.DMA,
        ],
    )
    def gather_sum(idx_hbm, table_hbm, feat_hbm, idx_v, bufs, feat_v, s0, s1):
        sems = (s0, s1)
        nc = 2
        wid = lax.axis_index("s") * nc + lax.axis_index("c")
        base = wid * pairs_per_w
        pltpu.sync_copy(idx_hbm.at[pl.ds(base, pairs_per_w)], idx_v)

        def issue(pair, h):
            pltpu.async_copy(table_hbm.at[idx_v.at[pair]], bufs.at[h],
                             sems[h])

        issue(0, 0)
        issue(1, 1)

        def step(i, carry):
            for h in range(2):
                pair = 2 * i + h
                pltpu.make_async_copy(table_hbm.at[pl.ds(0, C2)],
                                      bufs.at[h], sems[h]).wait()
                zero = jnp.zeros((_L,), jnp.float32)
                for half in range(2):
                    def acc_body(r, acc, h=h):
                        return tuple(acc[j] + bufs[h, r, pl.ds(j * _L, _L)]
                                     for j in range(NJ))

                    acc = plsc.parallel_loop(half * C, (half + 1) * C, 1,
                                             unroll=8,
                                             carry=(zero,) * NJ)(acc_body)
                    row = 2 * pair + half
                    for j in range(NJ):
                        feat_v[row, pl.ds(j * _L, _L)] = acc[j]

                @pl.when(pair < pairs_per_w - 2)
                def _(pair=pair, h=h):
                    issue(pair + 2, h)
            return carry

        lax.fori_loop(0, pairs_per_w // 2, step, 0)
        pltpu.sync_copy(feat_v, feat_hbm.at[pl.ds(wid * rows_per_w,
                                                  rows_per_w)])

    return gather_sum


def _linear_body(x_ref, w_ref, b_ref, o_ref):
    o_ref[...] = (
        jnp.dot(x_ref[...], w_ref[...], preferred_element_type=jnp.float32)
        + b_ref[...]
    )


def kernel(text, text_len, table, W, b):
    del text_len
    B, S = text.shape
    V, D = table.shape
    NC = W.shape[0]
    C = S + (-S) % (8 * _L)
    table_rm, H, NR = _row_major_table(table.T, V, D)
    remapped = jnp.where(text < V // 2, 2 * text, 2 * (text - H) + 1)
    idx = jnp.pad(remapped, ((0, 0), (0, C - S))).reshape(B // 2, 2 * C)
    table_lin = table_rm.reshape(2 * NR, D)
    info = plsc.get_sparse_core_info()
    NW = info.num_cores * info.num_subcores
    feat = _make_gather_sum(B, C, 2 * NR, D, NW)(idx, table_lin)
    out = pl.pallas_call(
        _linear_body,
        out_shape=jax.ShapeDtypeStruct((B, NC), jnp.float32),
    )(feat, W.T, b.reshape(1, NC))
    return out
